# baseline (speedup 1.0000x reference)
import functools

import jax
import jax.numpy as jnp
from jax import lax
from jax.experimental import pallas as pl
from jax.experimental.pallas import tpu as pltpu
from jax.experimental.pallas import tpu_sc as plsc

N = 10000
D = 128
E = 320000
NC = 2
NS = 16
K = 128
EPT = -(-E // (K * NS)) * K
EPAD = EPT * NS
CHUNKS = EPT // K
RPT = 640
ACC_N = RPT * NS
LANES = 16
CG = D // LANES


def _lane_bcast(v16, j):
  idx = jnp.full((LANES,), j, jnp.int32)
  dn = lax.GatherDimensionNumbers(
      offset_dims=(), collapsed_slice_dims=(0,), start_index_map=(0,))
  return lax.gather(v16, idx[:, None], dn, (1,),
                    mode=lax.GatherScatterMode.PROMISE_IN_BOUNDS)


def _seg_body(smsg, tmsg, row_h, col_h, ev_h, zeros_h, aggt_h, aggs_h,
              gidx_v, sidx_v, ev_v, rows_v, acc_sh):
  c = lax.axis_index("c")
  s = lax.axis_index("s")

  def run_dir(msg_h, g_h, sc_h, out_h):
    pltpu.sync_copy(zeros_h, acc_sh.at[pl.ds(s * RPT, RPT)])
    plsc.subcore_barrier()
    base = s * EPT

    def chunk(i, carry):
      off = base + i * K
      pltpu.sync_copy(g_h.at[pl.ds(off, K)], gidx_v)
      pltpu.sync_copy(sc_h.at[pl.ds(off, K)], sidx_v)
      pltpu.sync_copy(ev_h.at[pl.ds(off, K)], ev_v)
      pltpu.sync_copy(msg_h.at[gidx_v], rows_v)

      def scale(g, carry2):
        r0 = g * LANES
        ev16 = ev_v[pl.ds(r0, LANES)]
        for j in range(LANES):
          evb = _lane_bcast(ev16, j)
          for q in range(CG):
            sl = (r0 + j, pl.ds(q * LANES, LANES))
            rows_v[sl] = rows_v[sl] * evb
        return carry2

      lax.fori_loop(0, K // LANES, scale, 0)
      pltpu.sync_copy(rows_v, acc_sh.at[sidx_v], add=True)
      return carry

    lax.fori_loop(0, CHUNKS, chunk, 0)
    plsc.subcore_barrier()

    @pl.when(s < NS - 1)
    def _():
      pltpu.sync_copy(acc_sh.at[pl.ds(s * RPT, RPT)],
                      out_h.at[pl.ds(s * RPT, RPT)])

    @pl.when(s == NS - 1)
    def _():
      last = N - (NS - 1) * RPT
      pltpu.sync_copy(acc_sh.at[pl.ds((NS - 1) * RPT, last)],
                      out_h.at[pl.ds((NS - 1) * RPT, last)])

  @pl.when(c == 0)
  def _():
    run_dir(smsg, col_h, row_h, aggt_h)

  @pl.when(c == 1)
  def _():
    run_dir(tmsg, row_h, col_h, aggs_h)


_seg = pl.kernel(
    _seg_body,
    out_type=(jax.ShapeDtypeStruct((N, D), jnp.float32),
              jax.ShapeDtypeStruct((N, D), jnp.float32)),
    mesh=plsc.VectorSubcoreMesh(core_axis_name="c", subcore_axis_name="s",
                                num_cores=NC, num_subcores=NS),
    scratch_types=(
        pltpu.VMEM((K,), jnp.int32),
        pltpu.VMEM((K,), jnp.int32),
        pltpu.VMEM((K,), jnp.float32),
        pltpu.VMEM((K, D), jnp.float32),
        pltpu.VMEM_SHARED((ACC_N, D), jnp.float32),
    ),
)

_BR = 1000


def _proj_body(xs, xt, ws, wt, so, to):
  so[...] = jnp.dot(xs[...], ws[...], preferred_element_type=jnp.float32)
  to[...] = jnp.dot(xt[...], wt[...], preferred_element_type=jnp.float32)


_proj = pl.pallas_call(
    _proj_body,
    grid=(N // _BR,),
    in_specs=[pl.BlockSpec((_BR, D), lambda i: (i, 0)),
              pl.BlockSpec((_BR, D), lambda i: (i, 0)),
              pl.BlockSpec((D, D), lambda i: (0, 0)),
              pl.BlockSpec((D, D), lambda i: (0, 0))],
    out_specs=[pl.BlockSpec((_BR, D), lambda i: (i, 0)),
               pl.BlockSpec((_BR, D), lambda i: (i, 0))],
    out_shape=[jax.ShapeDtypeStruct((N, D), jnp.float32),
               jax.ShapeDtypeStruct((N, D), jnp.float32)],
)


def _upd_body(ags, agt, Ws, bs, Wt, bt, gma, bta, os_, ot_):
  def f(a, W, b):
    h = jnp.maximum(jnp.dot(a, W, preferred_element_type=jnp.float32) + b, 0.0)
    mu = jnp.mean(h, axis=-1, keepdims=True)
    var = jnp.mean((h - mu) ** 2, axis=-1, keepdims=True)
    y = (h - mu) / jnp.sqrt(var + 1e-5) * gma[...] + bta[...]
    return jnp.maximum(y, 0.0)
  os_[...] = f(ags[...], Ws[...], bs[...])
  ot_[...] = f(agt[...], Wt[...], bt[...])


_upd = pl.pallas_call(
    _upd_body,
    grid=(N // _BR,),
    in_specs=[pl.BlockSpec((_BR, D), lambda i: (i, 0)),
              pl.BlockSpec((_BR, D), lambda i: (i, 0)),
              pl.BlockSpec((D, D), lambda i: (0, 0)),
              pl.BlockSpec((1, D), lambda i: (0, 0)),
              pl.BlockSpec((D, D), lambda i: (0, 0)),
              pl.BlockSpec((1, D), lambda i: (0, 0)),
              pl.BlockSpec((1, D), lambda i: (0, 0)),
              pl.BlockSpec((1, D), lambda i: (0, 0))],
    out_specs=[pl.BlockSpec((_BR, D), lambda i: (i, 0)),
               pl.BlockSpec((_BR, D), lambda i: (i, 0))],
    out_shape=[jax.ShapeDtypeStruct((N, D), jnp.float32),
               jax.ShapeDtypeStruct((N, D), jnp.float32)],
)


def kernel(x_source, x_target, edge_index, edge_values, w_s, w_t, w_s_cci,
           w_t_cci, W_src_agg, b_src_agg, W_tgt_agg, b_tgt_agg, ln_gamma,
           ln_beta):
  row = edge_index[0].astype(jnp.int32)
  col = edge_index[1].astype(jnp.int32)
  pad = EPAD - E
  row = jnp.concatenate([row, jnp.zeros((pad,), jnp.int32)])
  col = jnp.concatenate([col, jnp.zeros((pad,), jnp.int32)])
  ev = jnp.concatenate([edge_values, jnp.zeros((pad,), jnp.float32)])
  zeros = jnp.zeros((RPT, D), jnp.float32)

  s_msg, t_msg = _proj(x_source, x_target, w_s, w_t)
  agg_t, agg_s = _seg(s_msg, t_msg, row, col, ev, zeros)
  out_source, out_target = _upd(
      agg_s, agg_t, W_src_agg, b_src_agg.reshape(1, D), W_tgt_agg,
      b_tgt_agg.reshape(1, D), ln_gamma.reshape(1, D), ln_beta.reshape(1, D))
  return out_source, out_target


# speedup vs baseline: 1.1637x; 1.1637x over previous
import functools

import jax
import jax.numpy as jnp
from jax import lax
from jax.experimental import pallas as pl
from jax.experimental.pallas import tpu as pltpu
from jax.experimental.pallas import tpu_sc as plsc

N = 10000
D = 128
E = 320000
NC = 2
NS = 16
K = 128
CHUNKS = 158
EPT = CHUNKS * K
EPAD = EPT * NS
RPT = 640
ACC_N = RPT * NS
LANES = 16
CG = D // LANES


def _lane_bcast(v16, j):
  idx = jnp.full((LANES,), j, jnp.int32)
  dn = lax.GatherDimensionNumbers(
      offset_dims=(), collapsed_slice_dims=(0,), start_index_map=(0,))
  return lax.gather(v16, idx[:, None], dn, (1,),
                    mode=lax.GatherScatterMode.PROMISE_IN_BOUNDS)


def _seg_body(smsg, tmsg, row_h, col_h, ev_h, zeros_h, aggt_h, aggs_h,
              gidx_v, sidx_v, ev_v, rows_v, isem0, isem1, acc_sh):
  c = lax.axis_index("c")
  s = lax.axis_index("s")
  isems = (isem0, isem1)

  def run_dir(msg_h, g_h, sc_h, out_h):
    pltpu.sync_copy(zeros_h, acc_sh.at[pl.ds(s * RPT, RPT)])
    plsc.subcore_barrier()
    base = s * EPT

    def start_idx(i, b):
      off = base + i * K
      pltpu.async_copy(g_h.at[pl.ds(off, K)], gidx_v.at[b], isems[b])
      pltpu.async_copy(sc_h.at[pl.ds(off, K)], sidx_v.at[b], isems[b])
      pltpu.async_copy(ev_h.at[pl.ds(off, K)], ev_v.at[b], isems[b])

    def wait_idx(b):
      pltpu.make_async_copy(g_h.at[pl.ds(0, K)], gidx_v.at[b],
                            isems[b]).wait()
      pltpu.make_async_copy(sc_h.at[pl.ds(0, K)], sidx_v.at[b],
                            isems[b]).wait()
      pltpu.make_async_copy(ev_h.at[pl.ds(0, K)], ev_v.at[b], isems[b]).wait()

    def process(i, b):
      pltpu.sync_copy(msg_h.at[gidx_v.at[b]], rows_v)

      def scale(g, carry2):
        r0 = g * LANES
        ev16 = ev_v[b, pl.ds(r0, LANES)]
        for j in range(LANES):
          evb = _lane_bcast(ev16, j)
          for q in range(CG):
            sl = (r0 + j, pl.ds(q * LANES, LANES))
            rows_v[sl] = rows_v[sl] * evb
        return carry2

      lax.fori_loop(0, K // LANES, scale, 0)
      pltpu.sync_copy(rows_v, acc_sh.at[sidx_v.at[b]], add=True)

    start_idx(0, 0)

    def pair(j2, carry):
      for b in range(2):
        i = j2 * 2 + b
        wait_idx(b)

        @pl.when(i + 1 < CHUNKS)
        def _():
          start_idx(i + 1, 1 - b)

        process(i, b)
      return carry

    lax.fori_loop(0, CHUNKS // 2, pair, 0)
    plsc.subcore_barrier()

    @pl.when(s < NS - 1)
    def _():
      pltpu.sync_copy(acc_sh.at[pl.ds(s * RPT, RPT)],
                      out_h.at[pl.ds(s * RPT, RPT)])

    @pl.when(s == NS - 1)
    def _():
      last = N - (NS - 1) * RPT
      pltpu.sync_copy(acc_sh.at[pl.ds((NS - 1) * RPT, last)],
                      out_h.at[pl.ds((NS - 1) * RPT, last)])

  @pl.when(c == 0)
  def _():
    run_dir(smsg, col_h, row_h, aggt_h)

  @pl.when(c == 1)
  def _():
    run_dir(tmsg, row_h, col_h, aggs_h)


_seg = pl.kernel(
    _seg_body,
    out_type=(jax.ShapeDtypeStruct((N, D), jnp.float32),
              jax.ShapeDtypeStruct((N, D), jnp.float32)),
    mesh=plsc.VectorSubcoreMesh(core_axis_name="c", subcore_axis_name="s",
                                num_cores=NC, num_subcores=NS),
    scratch_types=(
        pltpu.VMEM((2, K), jnp.int32),
        pltpu.VMEM((2, K), jnp.int32),
        pltpu.VMEM((2, K), jnp.float32),
        pltpu.VMEM((K, D), jnp.float32),
        pltpu.SemaphoreType.DMA,
        pltpu.SemaphoreType.DMA,
        pltpu.VMEM_SHARED((ACC_N, D), jnp.float32),
    ),
)

_BR = 1000


def _proj_body(xs, xt, ws, wt, so, to):
  so[...] = jnp.dot(xs[...], ws[...], preferred_element_type=jnp.float32)
  to[...] = jnp.dot(xt[...], wt[...], preferred_element_type=jnp.float32)


_proj = pl.pallas_call(
    _proj_body,
    grid=(N // _BR,),
    in_specs=[pl.BlockSpec((_BR, D), lambda i: (i, 0)),
              pl.BlockSpec((_BR, D), lambda i: (i, 0)),
              pl.BlockSpec((D, D), lambda i: (0, 0)),
              pl.BlockSpec((D, D), lambda i: (0, 0))],
    out_specs=[pl.BlockSpec((_BR, D), lambda i: (i, 0)),
               pl.BlockSpec((_BR, D), lambda i: (i, 0))],
    out_shape=[jax.ShapeDtypeStruct((N, D), jnp.float32),
               jax.ShapeDtypeStruct((N, D), jnp.float32)],
)


def _upd_body(ags, agt, Ws, bs, Wt, bt, gma, bta, os_, ot_):
  def f(a, W, b):
    h = jnp.maximum(jnp.dot(a, W, preferred_element_type=jnp.float32) + b, 0.0)
    mu = jnp.mean(h, axis=-1, keepdims=True)
    var = jnp.mean((h - mu) ** 2, axis=-1, keepdims=True)
    y = (h - mu) / jnp.sqrt(var + 1e-5) * gma[...] + bta[...]
    return jnp.maximum(y, 0.0)
  os_[...] = f(ags[...], Ws[...], bs[...])
  ot_[...] = f(agt[...], Wt[...], bt[...])


_upd = pl.pallas_call(
    _upd_body,
    grid=(N // _BR,),
    in_specs=[pl.BlockSpec((_BR, D), lambda i: (i, 0)),
              pl.BlockSpec((_BR, D), lambda i: (i, 0)),
              pl.BlockSpec((D, D), lambda i: (0, 0)),
              pl.BlockSpec((1, D), lambda i: (0, 0)),
              pl.BlockSpec((D, D), lambda i: (0, 0)),
              pl.BlockSpec((1, D), lambda i: (0, 0)),
              pl.BlockSpec((1, D), lambda i: (0, 0)),
              pl.BlockSpec((1, D), lambda i: (0, 0))],
    out_specs=[pl.BlockSpec((_BR, D), lambda i: (i, 0)),
               pl.BlockSpec((_BR, D), lambda i: (i, 0))],
    out_shape=[jax.ShapeDtypeStruct((N, D), jnp.float32),
               jax.ShapeDtypeStruct((N, D), jnp.float32)],
)


def kernel(x_source, x_target, edge_index, edge_values, w_s, w_t, w_s_cci,
           w_t_cci, W_src_agg, b_src_agg, W_tgt_agg, b_tgt_agg, ln_gamma,
           ln_beta):
  row = edge_index[0].astype(jnp.int32)
  col = edge_index[1].astype(jnp.int32)
  pad = EPAD - E
  row = jnp.concatenate([row, jnp.zeros((pad,), jnp.int32)])
  col = jnp.concatenate([col, jnp.zeros((pad,), jnp.int32)])
  ev = jnp.concatenate([edge_values, jnp.zeros((pad,), jnp.float32)])
  zeros = jnp.zeros((RPT, D), jnp.float32)

  s_msg, t_msg = _proj(x_source, x_target, w_s, w_t)
  agg_t, agg_s = _seg(s_msg, t_msg, row, col, ev, zeros)
  out_source, out_target = _upd(
      agg_s, agg_t, W_src_agg, b_src_agg.reshape(1, D), W_tgt_agg,
      b_tgt_agg.reshape(1, D), ln_gamma.reshape(1, D), ln_beta.reshape(1, D))
  return out_source, out_target
